# baseline (speedup 1.0000x reference)
import functools

import jax
import jax.numpy as jnp
from jax import lax
from jax.experimental import pallas as pl
from jax.experimental.pallas import tpu as pltpu
from jax.experimental.pallas import tpu_sc as plsc

_T = 4
_N = 10000
_E = 320000
_DIN = 128
_H = 48
_NC = 2
_NS = 16
_NW = _NC * _NS
_EPW = _E // _NW
_W = 128
_CH = 80
_PAD = _CH * _W - _EPW
_NP = 10240
_ROWS = _NP // _NS
_D1 = 64
_BN = 2000
_NB = _N // _BN


def _segsum_sc(xp, srcr, dstr, zrows, d):
    mesh = plsc.VectorSubcoreMesh(core_axis_name="c", subcore_axis_name="s")

    @functools.partial(
        pl.kernel,
        out_type=jax.ShapeDtypeStruct((_T, _NC, _NP, d), jnp.float32),
        mesh=mesh,
        compiler_params=pltpu.CompilerParams(use_tc_tiling_on_sc=False),
        scratch_types=[
            pltpu.VMEM((_CH, _W), jnp.int32),
            pltpu.VMEM((_CH, _W), jnp.int32),
            pltpu.VMEM((_W, d), jnp.float32),
            pltpu.VMEM((_W, d), jnp.float32),
            pltpu.VMEM_SHARED((_NP, d), jnp.float32),
            pltpu.SemaphoreType.DMA,
            pltpu.SemaphoreType.DMA,
            pltpu.SemaphoreType.DMA,
            pltpu.SemaphoreType.DMA,
        ],
    )
    def k(xp_hbm, src_hbm, dst_hbm, z_hbm, out_hbm, src_v, dst_v,
          rows_a, rows_b, acc_sh, gsa, gsb, ssa, ssb):
        c = lax.axis_index("c")
        s = lax.axis_index("s")
        wid = c * _NS + s
        for t in range(_T):
            pltpu.sync_copy(src_hbm.at[t].at[wid], src_v)
            pltpu.sync_copy(dst_hbm.at[t].at[wid], dst_v)
            pltpu.sync_copy(z_hbm, acc_sh.at[pl.ds(s * _ROWS, _ROWS)])
            plsc.subcore_barrier()
            xp_t = xp_hbm.at[t]

            def g(j, buf, sem):
                pltpu.async_copy(xp_t.at[src_v.at[j]], buf, sem)

            def sc(j, buf, sem):
                pltpu.async_copy(buf, acc_sh.at[dst_v.at[j]], sem, add=True)

            def gwait(buf, sem):
                pltpu.make_async_copy(xp_t.at[src_v.at[0]], buf, sem).wait()

            def swait(buf, sem):
                pltpu.make_async_copy(buf, acc_sh.at[dst_v.at[0]], sem).wait()

            g(0, rows_a, gsa)
            g(1, rows_b, gsb)

            @pl.loop(0, _CH // 2 - 1)
            def _(i):
                j0 = 2 * i
                gwait(rows_a, gsa)
                sc(j0, rows_a, ssa)
                swait(rows_a, ssa)
                g(j0 + 2, rows_a, gsa)
                gwait(rows_b, gsb)
                sc(j0 + 1, rows_b, ssb)
                swait(rows_b, ssb)
                g(j0 + 3, rows_b, gsb)

            gwait(rows_a, gsa)
            sc(_CH - 2, rows_a, ssa)
            swait(rows_a, ssa)
            gwait(rows_b, gsb)
            sc(_CH - 1, rows_b, ssb)
            swait(rows_b, ssb)

            plsc.subcore_barrier()
            pltpu.sync_copy(
                acc_sh.at[pl.ds(s * _ROWS, _ROWS)],
                out_hbm.at[t].at[c].at[pl.ds(s * _ROWS, _ROWS)],
            )

    return k(xp, srcr, dstr, zrows)


def _tc_pre(x, wn1t64, ws1t, b1row):
    def body(x_ref, wn_ref, ws_ref, b_ref, o1_ref, o2_ref):
        xb = x_ref[0]
        p64 = jnp.dot(xb, wn_ref[...], preferred_element_type=jnp.float32)
        col = lax.broadcasted_iota(jnp.int32, (_BN, _D1), 1)
        o1_ref[0] = jnp.where(col == _H, 1.0, p64)
        o2_ref[0] = (
            jnp.dot(xb, ws_ref[...], preferred_element_type=jnp.float32)
            + b_ref[...]
        )

    return pl.pallas_call(
        body,
        grid=(_T, _NB),
        in_specs=[
            pl.BlockSpec((1, _BN, _DIN), lambda t, n: (t, n, 0)),
            pl.BlockSpec((_DIN, _D1), lambda t, n: (0, 0)),
            pl.BlockSpec((_DIN, _H), lambda t, n: (0, 0)),
            pl.BlockSpec((1, _H), lambda t, n: (0, 0)),
        ],
        out_specs=[
            pl.BlockSpec((1, _BN, _D1), lambda t, n: (t, n, 0)),
            pl.BlockSpec((1, _BN, _H), lambda t, n: (t, n, 0)),
        ],
        out_shape=[
            jax.ShapeDtypeStruct((_T, _N, _D1), jnp.float32),
            jax.ShapeDtypeStruct((_T, _N, _H), jnp.float32),
        ],
    )(x, wn1t64, ws1t, b1row)


def _tc_mid(part1, xs1b, wn2t, ws2t, b2row):
    def body(p_ref, xs_ref, wn_ref, ws_ref, b_ref, o1_ref, o2_ref):
        srow = p_ref[0, 0] + p_ref[0, 1]
        deg = srow[:, _H:_H + 1]
        inv = 1.0 / jnp.maximum(deg, 1.0)
        h1 = jnp.maximum(xs_ref[0] + srow[:, :_H] * inv, 0.0)
        o1_ref[0] = jnp.dot(h1, wn_ref[...], preferred_element_type=jnp.float32)
        o2_ref[0] = (
            jnp.dot(h1, ws_ref[...], preferred_element_type=jnp.float32)
            + b_ref[...]
        )

    return pl.pallas_call(
        body,
        grid=(_T, _NB),
        in_specs=[
            pl.BlockSpec((1, _NC, _BN, _D1), lambda t, n: (t, 0, n, 0)),
            pl.BlockSpec((1, _BN, _H), lambda t, n: (t, n, 0)),
            pl.BlockSpec((_H, _H), lambda t, n: (0, 0)),

            pl.BlockSpec((_H, _H), lambda t, n: (0, 0)),
            pl.BlockSpec((1, _H), lambda t, n: (0, 0)),
        ],
        out_specs=[
            pl.BlockSpec((1, _BN, _H), lambda t, n: (t, n, 0)),
            pl.BlockSpec((1, _BN, _H), lambda t, n: (t, n, 0)),
        ],
        out_shape=[
            jax.ShapeDtypeStruct((_T, _N, _H), jnp.float32),
            jax.ShapeDtypeStruct((_T, _N, _H), jnp.float32),
        ],
    )(part1, xs1b, wn2t, ws2t, b2row)


def _tc_post(part2, hs2b, part1, smt, smbrow):
    def body(q_ref, hs_ref, p_ref, smt_ref, smb_ref, sp_ref, pool_ref):
        q = q_ref[0, 0] + q_ref[0, 1]
        deg = p_ref[0, 0][:, _H:_H + 1] + p_ref[0, 1][:, _H:_H + 1]
        inv = 1.0 / jnp.maximum(deg, 1.0)
        h2 = jnp.maximum(hs_ref[0] + q * inv, 0.0)
        sm = jax.nn.sigmoid(
            jnp.dot(h2, smt_ref[...], preferred_element_type=jnp.float32)
            + smb_ref[...]
        )
        sp_ref[0] = sm
        psum = jnp.sum(h2 * sm, axis=0, keepdims=True)[None]
        n = pl.program_id(1)

        @pl.when(n == 0)
        def _():
            pool_ref[...] = psum

        @pl.when(n > 0)
        def _():
            pool_ref[...] += psum

    return pl.pallas_call(
        body,
        grid=(_T, _NB),
        in_specs=[
            pl.BlockSpec((1, _NC, _BN, _H), lambda t, n: (t, 0, n, 0)),
            pl.BlockSpec((1, _BN, _H), lambda t, n: (t, n, 0)),
            pl.BlockSpec((1, _NC, _BN, _D1), lambda t, n: (t, 0, n, 0)),
            pl.BlockSpec((_H, _H), lambda t, n: (0, 0)),
            pl.BlockSpec((1, _H), lambda t, n: (0, 0)),
        ],
        out_specs=[
            pl.BlockSpec((1, _BN, _H), lambda t, n: (t, n, 0)),
            pl.BlockSpec((1, 1, _H), lambda t, n: (t, 0, 0)),
        ],
        out_shape=[
            jax.ShapeDtypeStruct((_T, _N, _H), jnp.float32),
            jax.ShapeDtypeStruct((_T, 1, _H), jnp.float32),
        ],
    )(part2, hs2b, part1, smt, smbrow)


def _tc_head(pooled, gf, gpt, gpbrow, wiht, whht, bihrow, bhhrow,
             attnt, attnb, tmt, tmb, reg1t, regb1row, reg2t, regb2, ws1):
    def body(pool_ref, gf_ref, gpt_ref, gpb_ref, wih_ref, whh_ref, bih_ref,
             bhh_ref, at_ref, ab_ref, tmt_ref, tmb_ref, r1_ref, rb1_ref,
             r2_ref, rb2_ref, ws1_ref,
             pred_ref, w_ref, cs_ref, tm_ref):
        pooled_mean = pool_ref[...] * (1.0 / _N)
        gs = jnp.maximum(
            jnp.dot(gf_ref[...], gpt_ref[...],
                    preferred_element_type=jnp.float32) + gpb_ref[...],
            0.0,
        )
        seq = jnp.concatenate([pooled_mean, gs], axis=1)
        h = jnp.zeros((1, _H), jnp.float32)
        outs = []
        for t in range(_T):
            gi = jnp.dot(seq[t:t + 1], wih_ref[...],
                         preferred_element_type=jnp.float32) + bih_ref[...]
            gh = jnp.dot(h, whh_ref[...],
                         preferred_element_type=jnp.float32) + bhh_ref[...]
            r = jax.nn.sigmoid(gi[:, :_H] + gh[:, :_H])
            z = jax.nn.sigmoid(gi[:, _H:2 * _H] + gh[:, _H:2 * _H])
            nn = jnp.tanh(gi[:, 2 * _H:] + r * gh[:, 2 * _H:])
            h = (1.0 - z) * nn + z * h
            outs.append(h)
        gru = jnp.concatenate(outs, axis=0)
        tmask = jax.nn.sigmoid(
            jnp.dot(gru, tmt_ref[...], preferred_element_type=jnp.float32)
            + tmb_ref[...]
        )
        scores = (
            jnp.dot(gru, at_ref[...], preferred_element_type=jnp.float32)
            + ab_ref[...]
        )
        m = jnp.max(scores)
        e = jnp.exp(scores - m)
        w = e / jnp.sum(e)
        context = jnp.sum(gru * w * tmask, axis=0, keepdims=True)
        hid = jnp.maximum(
            jnp.dot(context, r1_ref[...], preferred_element_type=jnp.float32)
            + rb1_ref[...],
            0.0,
        )
        pred_ref[...] = (
            jnp.dot(hid, r2_ref[...], preferred_element_type=jnp.float32)
            + rb2_ref[...]
        )
        w_ref[...] = w
        tm_ref[...] = tmask
        wn = jnp.mean(jnp.abs(ws1_ref[...]), axis=0, keepdims=True)
        col = lax.broadcasted_iota(jnp.int32, (1, _DIN), 1)
        v = jnp.where(col < 17, wn, -jnp.inf)
        vals = []
        for _ in range(8):
            mk = jnp.max(v)
            vals.append(jnp.reshape(mk, (1, 1)))
            cand = jnp.where(v == mk, col, 10**9)
            jmin = jnp.min(cand)
            v = jnp.where(col == jmin, -jnp.inf, v)
        cs_ref[...] = jnp.concatenate(vals, axis=1)

    return pl.pallas_call(
        body,
        out_shape=[
            jax.ShapeDtypeStruct((1, 1), jnp.float32),
            jax.ShapeDtypeStruct((_T, 1), jnp.float32),
            jax.ShapeDtypeStruct((1, 8), jnp.float32),
            jax.ShapeDtypeStruct((_T, 1), jnp.float32),
        ],
    )(pooled, gf, gpt, gpbrow, wiht, whht, bihrow, bhhrow, attnt, attnb,
      tmt, tmb, reg1t, regb1row, reg2t, regb2, ws1)


def kernel(node_features, edge_index, graph_features,
           conv1_Ws, conv1_bs, conv1_Wn, conv1_bn,
           conv2_Ws, conv2_bs, conv2_Wn, conv2_bn,
           gp_W, gp_b, gru_Wih, gru_Whh, gru_bih, gru_bhh,
           attn_W, attn_b, sm_W, sm_b, tm_W, tm_b,
           reg_W1, reg_b1, reg_W2, reg_b2):
    f32 = jnp.float32
    src = jnp.pad(
        edge_index[:, 0, :].reshape(_T, _NW, _EPW),
        ((0, 0), (0, 0), (0, _PAD)), constant_values=0,
    ).reshape(_T, _NW, _CH, _W)
    padrow = (_N + (jnp.arange(_NW, dtype=jnp.int32) % _NS))[None, :, None]
    dst = jnp.concatenate(
        [edge_index[:, 1, :].reshape(_T, _NW, _EPW),
         jnp.broadcast_to(padrow, (_T, _NW, _PAD))], axis=2,
    ).reshape(_T, _NW, _CH, _W)
    z64 = jnp.zeros((_ROWS, _D1), f32)
    z48 = jnp.zeros((_ROWS, _H), f32)

    wn1t64 = jnp.zeros((_DIN, _D1), f32).at[:, :_H].set(conv1_Wn.T)
    ws1t = conv1_Ws.T
    b1row = (conv1_bs + conv1_bn).reshape(1, _H)
    wn2t = conv2_Wn.T
    ws2t = conv2_Ws.T
    b2row = (conv2_bs + conv2_bn).reshape(1, _H)

    xp1, xs1b = _tc_pre(node_features, wn1t64, ws1t, b1row)
    part1 = _segsum_sc(xp1, src, dst, z64, _D1)
    hp2, hs2b = _tc_mid(part1, xs1b, wn2t, ws2t, b2row)
    part2 = _segsum_sc(hp2, src, dst, z48, _H)
    sp_masks, pooled = _tc_post(part2, hs2b, part1, sm_W.T, sm_b.reshape(1, _H))

    pred, w, cs, tm = _tc_head(
        pooled.reshape(_T, _H), graph_features, gp_W.T, gp_b.reshape(1, _H),
        gru_Wih.T, gru_Whh.T, gru_bih.reshape(1, 3 * _H),
        gru_bhh.reshape(1, 3 * _H),
        attn_W.T, attn_b.reshape(1, 1), tm_W.T, tm_b.reshape(1, 1),
        reg_W1.T, reg_b1.reshape(1, _H), reg_W2.T, reg_b2.reshape(1, 1),
        conv1_Ws,
    )
    return (pred.reshape(()), w.reshape(_T), cs.reshape(8), sp_masks,
            tm.reshape(_T))


# speedup vs baseline: 1.6212x; 1.6212x over previous
import functools

import jax
import jax.numpy as jnp
from jax import lax
from jax.experimental import pallas as pl
from jax.experimental.pallas import tpu as pltpu
from jax.experimental.pallas import tpu_sc as plsc

_T = 4
_N = 10000
_E = 320000
_DIN = 128
_H = 48
_NC = 2
_NS = 16
_NW = _NC * _NS
_EPW = _E // _NW
_W = 80
_CH = _EPW // _W
_NP = 10240
_ROWS = _NP // _NS
_D1 = 64
_BN = 2000
_NB = _N // _BN


def _segsum_sc(xp, srcr, dstr, zrows, d):
    mesh = plsc.VectorSubcoreMesh(core_axis_name="c", subcore_axis_name="s")

    @functools.partial(
        pl.kernel,
        out_type=jax.ShapeDtypeStruct((_T, 2 * _NC, _NP, d), jnp.float32),
        mesh=mesh,
        compiler_params=pltpu.CompilerParams(use_tc_tiling_on_sc=False),
        scratch_types=[
            pltpu.VMEM((_CH, _W), jnp.int32),
            pltpu.VMEM((_CH, _W), jnp.int32),
            pltpu.VMEM((_W, d), jnp.float32),
            pltpu.VMEM((_W, d), jnp.float32),
            pltpu.VMEM_SHARED((_NP, d), jnp.float32),
            pltpu.VMEM_SHARED((_NP, d), jnp.float32),
            pltpu.SemaphoreType.DMA,
            pltpu.SemaphoreType.DMA,
            pltpu.SemaphoreType.DMA,
            pltpu.SemaphoreType.DMA,
        ],
    )
    def k(xp_hbm, src_hbm, dst_hbm, z_hbm, out_hbm, src_v, dst_v,
          rows_a, rows_b, acc_a, acc_b, gsa, gsb, ssa, ssb):
        c = lax.axis_index("c")
        s = lax.axis_index("s")
        wid = c * _NS + s
        for t in range(_T):
            pltpu.sync_copy(src_hbm.at[t].at[wid], src_v)
            pltpu.sync_copy(dst_hbm.at[t].at[wid], dst_v)
            pltpu.sync_copy(z_hbm, acc_a.at[pl.ds(s * _ROWS, _ROWS)])
            pltpu.sync_copy(z_hbm, acc_b.at[pl.ds(s * _ROWS, _ROWS)])
            plsc.subcore_barrier()
            xp_t = xp_hbm.at[t]

            def g(j, buf, sem):
                pltpu.async_copy(xp_t.at[src_v.at[j]], buf, sem)

            def sc(j, buf, acc, sem):
                pltpu.async_copy(buf, acc.at[dst_v.at[j]], sem, add=True)

            def gwait(buf, sem):
                pltpu.make_async_copy(xp_t.at[src_v.at[0]], buf, sem).wait()

            def swait(buf, acc, sem):
                pltpu.make_async_copy(buf, acc.at[dst_v.at[0]], sem).wait()

            pltpu.sync_copy(xp_t.at[src_v.at[0]], rows_a)
            pltpu.sync_copy(rows_a, acc_a.at[dst_v.at[0]], add=True)
            g(1, rows_a, gsa)
            g(2, rows_b, gsb)

            @pl.loop(0, (_CH - 3) // 2)
            def _(i):
                j0 = 2 * i + 1
                gwait(rows_a, gsa)
                sc(j0, rows_a, acc_a, ssa)
                gwait(rows_b, gsb)
                sc(j0 + 1, rows_b, acc_b, ssb)
                swait(rows_a, acc_a, ssa)
                g(j0 + 2, rows_a, gsa)
                swait(rows_b, acc_b, ssb)
                g(j0 + 3, rows_b, gsb)

            gwait(rows_a, gsa)
            sc(_CH - 2, rows_a, acc_a, ssa)
            gwait(rows_b, gsb)
            sc(_CH - 1, rows_b, acc_b, ssb)
            swait(rows_a, acc_a, ssa)
            swait(rows_b, acc_b, ssb)

            plsc.subcore_barrier()
            pltpu.sync_copy(
                acc_a.at[pl.ds(s * _ROWS, _ROWS)],
                out_hbm.at[t].at[2 * c].at[pl.ds(s * _ROWS, _ROWS)],
            )
            pltpu.sync_copy(
                acc_b.at[pl.ds(s * _ROWS, _ROWS)],
                out_hbm.at[t].at[2 * c + 1].at[pl.ds(s * _ROWS, _ROWS)],
            )

    return k(xp, srcr, dstr, zrows)


def _tc_pre(x, wn1t64, ws1t, b1row):
    def body(x_ref, wn_ref, ws_ref, b_ref, o1_ref, o2_ref):
        xb = x_ref[0]
        p64 = jnp.dot(xb, wn_ref[...], preferred_element_type=jnp.float32)
        col = lax.broadcasted_iota(jnp.int32, (_BN, _D1), 1)
        o1_ref[0] = jnp.where(col == _H, 1.0, p64)
        o2_ref[0] = (
            jnp.dot(xb, ws_ref[...], preferred_element_type=jnp.float32)
            + b_ref[...]
        )

    return pl.pallas_call(
        body,
        grid=(_T, _NB),
        in_specs=[
            pl.BlockSpec((1, _BN, _DIN), lambda t, n: (t, n, 0)),
            pl.BlockSpec((_DIN, _D1), lambda t, n: (0, 0)),
            pl.BlockSpec((_DIN, _H), lambda t, n: (0, 0)),
            pl.BlockSpec((1, _H), lambda t, n: (0, 0)),
        ],
        out_specs=[
            pl.BlockSpec((1, _BN, _D1), lambda t, n: (t, n, 0)),
            pl.BlockSpec((1, _BN, _H), lambda t, n: (t, n, 0)),
        ],
        out_shape=[
            jax.ShapeDtypeStruct((_T, _N, _D1), jnp.float32),
            jax.ShapeDtypeStruct((_T, _N, _H), jnp.float32),
        ],
    )(x, wn1t64, ws1t, b1row)


def _tc_mid(part1, xs1b, wn2t, ws2t, b2row):
    def body(p_ref, xs_ref, wn_ref, ws_ref, b_ref, o1_ref, o2_ref):
        srow = (p_ref[0, 0] + p_ref[0, 1]) + (p_ref[0, 2] + p_ref[0, 3])
        deg = srow[:, _H:_H + 1]
        inv = 1.0 / jnp.maximum(deg, 1.0)
        h1 = jnp.maximum(xs_ref[0] + srow[:, :_H] * inv, 0.0)
        o1_ref[0] = jnp.dot(h1, wn_ref[...], preferred_element_type=jnp.float32)
        o2_ref[0] = (
            jnp.dot(h1, ws_ref[...], preferred_element_type=jnp.float32)
            + b_ref[...]
        )

    return pl.pallas_call(
        body,
        grid=(_T, _NB),
        in_specs=[
            pl.BlockSpec((1, 2 * _NC, _BN, _D1), lambda t, n: (t, 0, n, 0)),
            pl.BlockSpec((1, _BN, _H), lambda t, n: (t, n, 0)),
            pl.BlockSpec((_H, _H), lambda t, n: (0, 0)),

            pl.BlockSpec((_H, _H), lambda t, n: (0, 0)),
            pl.BlockSpec((1, _H), lambda t, n: (0, 0)),
        ],
        out_specs=[
            pl.BlockSpec((1, _BN, _H), lambda t, n: (t, n, 0)),
            pl.BlockSpec((1, _BN, _H), lambda t, n: (t, n, 0)),
        ],
        out_shape=[
            jax.ShapeDtypeStruct((_T, _N, _H), jnp.float32),
            jax.ShapeDtypeStruct((_T, _N, _H), jnp.float32),
        ],
    )(part1, xs1b, wn2t, ws2t, b2row)


def _tc_post(part2, hs2b, part1, smt, smbrow):
    def body(q_ref, hs_ref, p_ref, smt_ref, smb_ref, sp_ref, pool_ref):
        q = (q_ref[0, 0] + q_ref[0, 1]) + (q_ref[0, 2] + q_ref[0, 3])
        deg = ((p_ref[0, 0][:, _H:_H + 1] + p_ref[0, 1][:, _H:_H + 1])
               + (p_ref[0, 2][:, _H:_H + 1] + p_ref[0, 3][:, _H:_H + 1]))
        inv = 1.0 / jnp.maximum(deg, 1.0)
        h2 = jnp.maximum(hs_ref[0] + q * inv, 0.0)
        sm = jax.nn.sigmoid(
            jnp.dot(h2, smt_ref[...], preferred_element_type=jnp.float32)
            + smb_ref[...]
        )
        sp_ref[0] = sm
        psum = jnp.sum(h2 * sm, axis=0, keepdims=True)[None]
        n = pl.program_id(1)

        @pl.when(n == 0)
        def _():
            pool_ref[...] = psum

        @pl.when(n > 0)
        def _():
            pool_ref[...] += psum

    return pl.pallas_call(
        body,
        grid=(_T, _NB),
        in_specs=[
            pl.BlockSpec((1, 2 * _NC, _BN, _H), lambda t, n: (t, 0, n, 0)),
            pl.BlockSpec((1, _BN, _H), lambda t, n: (t, n, 0)),
            pl.BlockSpec((1, 2 * _NC, _BN, _D1), lambda t, n: (t, 0, n, 0)),
            pl.BlockSpec((_H, _H), lambda t, n: (0, 0)),
            pl.BlockSpec((1, _H), lambda t, n: (0, 0)),
        ],
        out_specs=[
            pl.BlockSpec((1, _BN, _H), lambda t, n: (t, n, 0)),
            pl.BlockSpec((1, 1, _H), lambda t, n: (t, 0, 0)),
        ],
        out_shape=[
            jax.ShapeDtypeStruct((_T, _N, _H), jnp.float32),
            jax.ShapeDtypeStruct((_T, 1, _H), jnp.float32),
        ],
    )(part2, hs2b, part1, smt, smbrow)


def _tc_head(pooled, gf, gpt, gpbrow, wiht, whht, bihrow, bhhrow,
             attnt, attnb, tmt, tmb, reg1t, regb1row, reg2t, regb2, ws1):
    def body(pool_ref, gf_ref, gpt_ref, gpb_ref, wih_ref, whh_ref, bih_ref,
             bhh_ref, at_ref, ab_ref, tmt_ref, tmb_ref, r1_ref, rb1_ref,
             r2_ref, rb2_ref, ws1_ref,
             pred_ref, w_ref, cs_ref, tm_ref):
        pooled_mean = pool_ref[...] * (1.0 / _N)
        gs = jnp.maximum(
            jnp.dot(gf_ref[...], gpt_ref[...],
                    preferred_element_type=jnp.float32) + gpb_ref[...],
            0.0,
        )
        seq = jnp.concatenate([pooled_mean, gs], axis=1)
        h = jnp.zeros((1, _H), jnp.float32)
        outs = []
        for t in range(_T):
            gi = jnp.dot(seq[t:t + 1], wih_ref[...],
                         preferred_element_type=jnp.float32) + bih_ref[...]
            gh = jnp.dot(h, whh_ref[...],
                         preferred_element_type=jnp.float32) + bhh_ref[...]
            r = jax.nn.sigmoid(gi[:, :_H] + gh[:, :_H])
            z = jax.nn.sigmoid(gi[:, _H:2 * _H] + gh[:, _H:2 * _H])
            nn = jnp.tanh(gi[:, 2 * _H:] + r * gh[:, 2 * _H:])
            h = (1.0 - z) * nn + z * h
            outs.append(h)
        gru = jnp.concatenate(outs, axis=0)
        tmask = jax.nn.sigmoid(
            jnp.dot(gru, tmt_ref[...], preferred_element_type=jnp.float32)
            + tmb_ref[...]
        )
        scores = (
            jnp.dot(gru, at_ref[...], preferred_element_type=jnp.float32)
            + ab_ref[...]
        )
        m = jnp.max(scores)
        e = jnp.exp(scores - m)
        w = e / jnp.sum(e)
        context = jnp.sum(gru * w * tmask, axis=0, keepdims=True)
        hid = jnp.maximum(
            jnp.dot(context, r1_ref[...], preferred_element_type=jnp.float32)
            + rb1_ref[...],
            0.0,
        )
        pred_ref[...] = (
            jnp.dot(hid, r2_ref[...], preferred_element_type=jnp.float32)
            + rb2_ref[...]
        )
        w_ref[...] = w
        tm_ref[...] = tmask
        wn = jnp.mean(jnp.abs(ws1_ref[...]), axis=0, keepdims=True)
        col = lax.broadcasted_iota(jnp.int32, (1, _DIN), 1)
        v = jnp.where(col < 17, wn, -jnp.inf)
        vals = []
        for _ in range(8):
            mk = jnp.max(v)
            vals.append(jnp.reshape(mk, (1, 1)))
            cand = jnp.where(v == mk, col, 10**9)
            jmin = jnp.min(cand)
            v = jnp.where(col == jmin, -jnp.inf, v)
        cs_ref[...] = jnp.concatenate(vals, axis=1)

    return pl.pallas_call(
        body,
        out_shape=[
            jax.ShapeDtypeStruct((1, 1), jnp.float32),
            jax.ShapeDtypeStruct((_T, 1), jnp.float32),
            jax.ShapeDtypeStruct((1, 8), jnp.float32),
            jax.ShapeDtypeStruct((_T, 1), jnp.float32),
        ],
    )(pooled, gf, gpt, gpbrow, wiht, whht, bihrow, bhhrow, attnt, attnb,
      tmt, tmb, reg1t, regb1row, reg2t, regb2, ws1)


def kernel(node_features, edge_index, graph_features,
           conv1_Ws, conv1_bs, conv1_Wn, conv1_bn,
           conv2_Ws, conv2_bs, conv2_Wn, conv2_bn,
           gp_W, gp_b, gru_Wih, gru_Whh, gru_bih, gru_bhh,
           attn_W, attn_b, sm_W, sm_b, tm_W, tm_b,
           reg_W1, reg_b1, reg_W2, reg_b2):
    f32 = jnp.float32
    src = edge_index[:, 0, :].reshape(_T, _NW, _CH, _W)
    dst = edge_index[:, 1, :].reshape(_T, _NW, _CH, _W)
    z64 = jnp.zeros((_ROWS, _D1), f32)
    z48 = jnp.zeros((_ROWS, _H), f32)

    wn1t64 = jnp.zeros((_DIN, _D1), f32).at[:, :_H].set(conv1_Wn.T)
    ws1t = conv1_Ws.T
    b1row = (conv1_bs + conv1_bn).reshape(1, _H)
    wn2t = conv2_Wn.T
    ws2t = conv2_Ws.T
    b2row = (conv2_bs + conv2_bn).reshape(1, _H)

    xp1, xs1b = _tc_pre(node_features, wn1t64, ws1t, b1row)
    part1 = _segsum_sc(xp1, src, dst, z64, _D1)
    hp2, hs2b = _tc_mid(part1, xs1b, wn2t, ws2t, b2row)
    part2 = _segsum_sc(hp2, src, dst, z48, _H)
    sp_masks, pooled = _tc_post(part2, hs2b, part1, sm_W.T, sm_b.reshape(1, _H))

    pred, w, cs, tm = _tc_head(
        pooled.reshape(_T, _H), graph_features, gp_W.T, gp_b.reshape(1, _H),
        gru_Wih.T, gru_Whh.T, gru_bih.reshape(1, 3 * _H),
        gru_bhh.reshape(1, 3 * _H),
        attn_W.T, attn_b.reshape(1, 1), tm_W.T, tm_b.reshape(1, 1),
        reg_W1.T, reg_b1.reshape(1, _H), reg_W2.T, reg_b2.reshape(1, 1),
        conv1_Ws,
    )
    return (pred.reshape(()), w.reshape(_T), cs.reshape(8), sp_masks,
            tm.reshape(_T))


# speedup vs baseline: 2.1718x; 1.3396x over previous
import functools

import jax
import jax.numpy as jnp
from jax import lax
from jax.experimental import pallas as pl
from jax.experimental.pallas import tpu as pltpu
from jax.experimental.pallas import tpu_sc as plsc

_T = 4
_N = 10000
_E = 320000
_DIN = 128
_H = 48
_NC = 2
_NS = 16
_NW = _NC * _NS
_EPW = _E // _NW
_W = 80
_CH = _EPW // _W
_NP = 10240
_ROWS = _NP // _NS
_D1 = 64
_BN = 2000
_NB = _N // _BN


def _segsum_sc(xp, srcr, dstr, zrows, d, tx, te):
    mesh = plsc.VectorSubcoreMesh(core_axis_name="c", subcore_axis_name="s")

    @functools.partial(
        pl.kernel,
        out_type=jax.ShapeDtypeStruct((_NC, _NP, d), jnp.float32),
        mesh=mesh,
        compiler_params=pltpu.CompilerParams(use_tc_tiling_on_sc=False),
        scratch_types=[
            pltpu.VMEM((_CH, _W), jnp.int32),
            pltpu.VMEM((_CH, _W), jnp.int32),
            pltpu.VMEM((_W, d), jnp.float32),
            pltpu.VMEM((_W, d), jnp.float32),
            pltpu.VMEM_SHARED((_NP, d), jnp.float32),
            pltpu.SemaphoreType.DMA,
            pltpu.SemaphoreType.DMA,
            pltpu.SemaphoreType.DMA,
            pltpu.SemaphoreType.DMA,
        ],
    )
    def k(xp_hbm, src_hbm, dst_hbm, z_hbm, out_hbm, src_v, dst_v,
          rows_a, rows_b, acc_sh, gsa, gsb, ssa, ssb):
        c = lax.axis_index("c")
        s = lax.axis_index("s")
        wid = c * _NS + s
        pltpu.sync_copy(src_hbm.at[te].at[wid], src_v)
        pltpu.sync_copy(dst_hbm.at[te].at[wid], dst_v)
        pltpu.sync_copy(z_hbm, acc_sh.at[pl.ds(s * _ROWS, _ROWS)])
        plsc.subcore_barrier()
        xp_t = xp_hbm.at[tx]

        def g(j, buf, sem):
            pltpu.async_copy(xp_t.at[src_v.at[j]], buf, sem)

        def sc(j, buf, sem):
            pltpu.async_copy(buf, acc_sh.at[dst_v.at[j]], sem, add=True)

        def gwait(buf, sem):
            pltpu.make_async_copy(xp_t.at[src_v.at[0]], buf, sem).wait()

        def swait(buf, sem):
            pltpu.make_async_copy(buf, acc_sh.at[dst_v.at[0]], sem).wait()

        pltpu.sync_copy(xp_t.at[src_v.at[0]], rows_a)
        pltpu.sync_copy(rows_a, acc_sh.at[dst_v.at[0]], add=True)
        g(1, rows_a, gsa)
        g(2, rows_b, gsb)

        @pl.loop(0, (_CH - 3) // 2)
        def _(i):
            j0 = 2 * i + 1
            gwait(rows_a, gsa)
            sc(j0, rows_a, ssa)
            swait(rows_a, ssa)
            g(j0 + 2, rows_a, gsa)
            gwait(rows_b, gsb)
            sc(j0 + 1, rows_b, ssb)
            swait(rows_b, ssb)
            g(j0 + 3, rows_b, gsb)

        gwait(rows_a, gsa)
        sc(_CH - 2, rows_a, ssa)
        swait(rows_a, ssa)
        gwait(rows_b, gsb)
        sc(_CH - 1, rows_b, ssb)
        swait(rows_b, ssb)

        plsc.subcore_barrier()
        pltpu.sync_copy(
            acc_sh.at[pl.ds(s * _ROWS, _ROWS)],
            out_hbm.at[c].at[pl.ds(s * _ROWS, _ROWS)],
        )

    return k(xp, srcr, dstr, zrows)


def _tc_pre(x, wn1t64, ws1t, b1row):
    def body(x_ref, wn_ref, ws_ref, b_ref, o1_ref, o2_ref):
        xb = x_ref[0]
        p64 = jnp.dot(xb, wn_ref[...], preferred_element_type=jnp.float32)
        col = lax.broadcasted_iota(jnp.int32, (_BN, _D1), 1)
        o1_ref[0] = jnp.where(col == _H, 1.0, p64)
        o2_ref[0] = (
            jnp.dot(xb, ws_ref[...], preferred_element_type=jnp.float32)
            + b_ref[...]
        )

    return pl.pallas_call(
        body,
        grid=(_T, _NB),
        in_specs=[
            pl.BlockSpec((1, _BN, _DIN), lambda t, n: (t, n, 0)),
            pl.BlockSpec((_DIN, _D1), lambda t, n: (0, 0)),
            pl.BlockSpec((_DIN, _H), lambda t, n: (0, 0)),
            pl.BlockSpec((1, _H), lambda t, n: (0, 0)),
        ],
        out_specs=[
            pl.BlockSpec((1, _BN, _D1), lambda t, n: (t, n, 0)),
            pl.BlockSpec((1, _BN, _H), lambda t, n: (t, n, 0)),
        ],
        out_shape=[
            jax.ShapeDtypeStruct((_T, _N, _D1), jnp.float32),
            jax.ShapeDtypeStruct((_T, _N, _H), jnp.float32),
        ],
    )(x, wn1t64, ws1t, b1row)


def _tc_mid(part1, xs1b, wn2t, ws2t, b2row, t):
    def body(p_ref, xs_ref, wn_ref, ws_ref, b_ref, o1_ref, o2_ref):
        srow = p_ref[0] + p_ref[1]
        deg = srow[:, _H:_H + 1]
        inv = 1.0 / jnp.maximum(deg, 1.0)
        h1 = jnp.maximum(xs_ref[0] + srow[:, :_H] * inv, 0.0)
        o1_ref[...] = jnp.dot(h1, wn_ref[...],
                              preferred_element_type=jnp.float32)
        o2_ref[...] = (
            jnp.dot(h1, ws_ref[...], preferred_element_type=jnp.float32)
            + b_ref[...]
        )

    return pl.pallas_call(
        body,
        grid=(_NB,),
        in_specs=[
            pl.BlockSpec((_NC, _BN, _D1), lambda n: (0, n, 0)),
            pl.BlockSpec((1, _BN, _H), lambda n: (t, n, 0)),
            pl.BlockSpec((_H, _H), lambda n: (0, 0)),
            pl.BlockSpec((_H, _H), lambda n: (0, 0)),
            pl.BlockSpec((1, _H), lambda n: (0, 0)),
        ],
        out_specs=[
            pl.BlockSpec((_BN, _H), lambda n: (n, 0)),
            pl.BlockSpec((_BN, _H), lambda n: (n, 0)),
        ],
        out_shape=[
            jax.ShapeDtypeStruct((_N, _H), jnp.float32),
            jax.ShapeDtypeStruct((_N, _H), jnp.float32),
        ],
    )(part1, xs1b, wn2t, ws2t, b2row)


def _tc_post(part2, hs2b, part1, smt, smbrow):
    def body(q_ref, hs_ref, p_ref, smt_ref, smb_ref, sp_ref, pool_ref):
        q = q_ref[0] + q_ref[1]
        deg = p_ref[0][:, _H:_H + 1] + p_ref[1][:, _H:_H + 1]
        inv = 1.0 / jnp.maximum(deg, 1.0)
        h2 = jnp.maximum(hs_ref[...] + q * inv, 0.0)
        sm = jax.nn.sigmoid(
            jnp.dot(h2, smt_ref[...], preferred_element_type=jnp.float32)
            + smb_ref[...]
        )
        sp_ref[...] = sm
        psum = jnp.sum(h2 * sm, axis=0, keepdims=True)
        n = pl.program_id(0)

        @pl.when(n == 0)
        def _():
            pool_ref[...] = psum

        @pl.when(n > 0)
        def _():
            pool_ref[...] += psum

    return pl.pallas_call(
        body,
        grid=(_NB,),
        in_specs=[
            pl.BlockSpec((_NC, _BN, _H), lambda n: (0, n, 0)),
            pl.BlockSpec((_BN, _H), lambda n: (n, 0)),
            pl.BlockSpec((_NC, _BN, _D1), lambda n: (0, n, 0)),
            pl.BlockSpec((_H, _H), lambda n: (0, 0)),
            pl.BlockSpec((1, _H), lambda n: (0, 0)),
        ],
        out_specs=[
            pl.BlockSpec((_BN, _H), lambda n: (n, 0)),
            pl.BlockSpec((1, _H), lambda n: (0, 0)),
        ],
        out_shape=[
            jax.ShapeDtypeStruct((_N, _H), jnp.float32),
            jax.ShapeDtypeStruct((1, _H), jnp.float32),
        ],
    )(part2, hs2b, part1, smt, smbrow)


def _tc_head(pooled, gf, gpt, gpbrow, wiht, whht, bihrow, bhhrow,
             attnt, attnb, tmt, tmb, reg1t, regb1row, reg2t, regb2, ws1):
    def body(pool_ref, gf_ref, gpt_ref, gpb_ref, wih_ref, whh_ref, bih_ref,
             bhh_ref, at_ref, ab_ref, tmt_ref, tmb_ref, r1_ref, rb1_ref,
             r2_ref, rb2_ref, ws1_ref,
             pred_ref, w_ref, cs_ref, tm_ref):
        pooled_mean = pool_ref[...] * (1.0 / _N)
        gs = jnp.maximum(
            jnp.dot(gf_ref[...], gpt_ref[...],
                    preferred_element_type=jnp.float32) + gpb_ref[...],
            0.0,
        )
        seq = jnp.concatenate([pooled_mean, gs], axis=1)
        h = jnp.zeros((1, _H), jnp.float32)
        outs = []
        for t in range(_T):
            gi = jnp.dot(seq[t:t + 1], wih_ref[...],
                         preferred_element_type=jnp.float32) + bih_ref[...]
            gh = jnp.dot(h, whh_ref[...],
                         preferred_element_type=jnp.float32) + bhh_ref[...]
            r = jax.nn.sigmoid(gi[:, :_H] + gh[:, :_H])
            z = jax.nn.sigmoid(gi[:, _H:2 * _H] + gh[:, _H:2 * _H])
            nn = jnp.tanh(gi[:, 2 * _H:] + r * gh[:, 2 * _H:])
            h = (1.0 - z) * nn + z * h
            outs.append(h)
        gru = jnp.concatenate(outs, axis=0)
        tmask = jax.nn.sigmoid(
            jnp.dot(gru, tmt_ref[...], preferred_element_type=jnp.float32)
            + tmb_ref[...]
        )
        scores = (
            jnp.dot(gru, at_ref[...], preferred_element_type=jnp.float32)
            + ab_ref[...]
        )
        m = jnp.max(scores)
        e = jnp.exp(scores - m)
        w = e / jnp.sum(e)
        context = jnp.sum(gru * w * tmask, axis=0, keepdims=True)
        hid = jnp.maximum(
            jnp.dot(context, r1_ref[...], preferred_element_type=jnp.float32)
            + rb1_ref[...],
            0.0,
        )
        pred_ref[...] = (
            jnp.dot(hid, r2_ref[...], preferred_element_type=jnp.float32)
            + rb2_ref[...]
        )
        w_ref[...] = w
        tm_ref[...] = tmask
        wn = jnp.mean(jnp.abs(ws1_ref[...]), axis=0, keepdims=True)
        col = lax.broadcasted_iota(jnp.int32, (1, _DIN), 1)
        v = jnp.where(col < 17, wn, -jnp.inf)
        vals = []
        for _ in range(8):
            mk = jnp.max(v)
            vals.append(jnp.reshape(mk, (1, 1)))
            cand = jnp.where(v == mk, col, 10**9)
            jmin = jnp.min(cand)
            v = jnp.where(col == jmin, -jnp.inf, v)
        cs_ref[...] = jnp.concatenate(vals, axis=1)

    return pl.pallas_call(
        body,
        out_shape=[
            jax.ShapeDtypeStruct((1, 1), jnp.float32),
            jax.ShapeDtypeStruct((_T, 1), jnp.float32),
            jax.ShapeDtypeStruct((1, 8), jnp.float32),
            jax.ShapeDtypeStruct((_T, 1), jnp.float32),
        ],
    )(pooled, gf, gpt, gpbrow, wiht, whht, bihrow, bhhrow, attnt, attnb,
      tmt, tmb, reg1t, regb1row, reg2t, regb2, ws1)


def kernel(node_features, edge_index, graph_features,
           conv1_Ws, conv1_bs, conv1_Wn, conv1_bn,
           conv2_Ws, conv2_bs, conv2_Wn, conv2_bn,
           gp_W, gp_b, gru_Wih, gru_Whh, gru_bih, gru_bhh,
           attn_W, attn_b, sm_W, sm_b, tm_W, tm_b,
           reg_W1, reg_b1, reg_W2, reg_b2):
    f32 = jnp.float32
    src = edge_index[:, 0, :].reshape(_T, _NW, _CH, _W)
    dst = edge_index[:, 1, :].reshape(_T, _NW, _CH, _W)
    z64 = jnp.zeros((_ROWS, _D1), f32)
    z48 = jnp.zeros((_ROWS, _H), f32)

    wn1t64 = jnp.zeros((_DIN, _D1), f32).at[:, :_H].set(conv1_Wn.T)
    ws1t = conv1_Ws.T
    b1row = (conv1_bs + conv1_bn).reshape(1, _H)
    wn2t = conv2_Wn.T
    ws2t = conv2_Ws.T
    b2row = (conv2_bs + conv2_bn).reshape(1, _H)
    smt = sm_W.T
    smbrow = sm_b.reshape(1, _H)

    xp1, xs1b = _tc_pre(node_features, wn1t64, ws1t, b1row)
    sp_list, pool_list = [], []
    for t in range(_T):
        part1 = _segsum_sc(xp1, src, dst, z64, _D1, t, t)
        hp2, hs2b = _tc_mid(part1, xs1b, wn2t, ws2t, b2row, t)
        part2 = _segsum_sc(hp2[None], src, dst, z48, _H, 0, t)
        sp_t, pool_t = _tc_post(part2, hs2b, part1, smt, smbrow)
        sp_list.append(sp_t)
        pool_list.append(pool_t)
    sp_masks = jnp.stack(sp_list, axis=0)
    pooled = jnp.concatenate(pool_list, axis=0)

    pred, w, cs, tm = _tc_head(
        pooled, graph_features, gp_W.T, gp_b.reshape(1, _H),
        gru_Wih.T, gru_Whh.T, gru_bih.reshape(1, 3 * _H),
        gru_bhh.reshape(1, 3 * _H),
        attn_W.T, attn_b.reshape(1, 1), tm_W.T, tm_b.reshape(1, 1),
        reg_W1.T, reg_b1.reshape(1, _H), reg_W2.T, reg_b2.reshape(1, 1),
        conv1_Ws,
    )
    return (pred.reshape(()), w.reshape(_T), cs.reshape(8), sp_masks,
            tm.reshape(_T))
